# baseline (speedup 1.0000x reference)
import jax
import jax.numpy as jnp
from jax import lax
from jax.experimental import pallas as pl
from jax.experimental.pallas import tpu as pltpu
from jax.experimental.pallas import tpu_sc as plsc

NBG = 2
N = 10000
FT = 128
HID = 128
E = 320000

NTILES = 16
EPT = E // NTILES
CH = 80
NCH = EPT // CH
NP = 10240
NHALF = NP // 2
ACC_R = NHALF + 128
ZPT = ACC_R // NTILES
FPT = NHALF // NTILES


def _mm_body(f_ref, s_ref, w_ref, o_ref):
    w = w_ref[0]
    o_ref[0] = jnp.dot(f_ref[0, 0], w, preferred_element_type=jnp.float32)
    o_ref[1] = jnp.dot(s_ref[0, 0], w, preferred_element_type=jnp.float32)


def _gcn_matmul(feature, shuf, w):
    return pl.pallas_call(
        _mm_body,
        grid=(NBG,),
        in_specs=[
            pl.BlockSpec((1, 1, N, FT), lambda g: (g, 0, 0, 0)),
            pl.BlockSpec((1, 1, N, FT), lambda g: (g, 0, 0, 0)),
            pl.BlockSpec((1, FT, HID), lambda g: (g, 0, 0)),
        ],
        out_specs=pl.BlockSpec((2, N, HID), lambda g: (g, 0, 0)),
        out_shape=jax.ShapeDtypeStruct((4, N, HID), jnp.float32),
    )(feature, shuf, w)


def _spmm_body(table_hbm, src_hbm, dst_hbm, zeros_hbm, out_hbm,
               src_v, dst_v, rows_v, accum, sem0, sem1, ssem0, ssem1):
    c = lax.axis_index("c")
    s = lax.axis_index("s")
    sems = (sem0, sem1)
    ssems = (ssem0, ssem1)

    for g in range(NBG):
        for t in range(2):
            pltpu.sync_copy(zeros_hbm, accum.at[pl.ds(s * ZPT, ZPT)])
            pltpu.sync_copy(src_hbm.at[g, t, s], src_v)
            pltpu.sync_copy(dst_hbm.at[c, g, s], dst_v)
            plsc.subcore_barrier()

            def _gather(k, b):
                return pltpu.make_async_copy(
                    table_hbm.at[src_v.at[k]], rows_v.at[b], sems[b])

            def _scatter_drain(b):
                pltpu.make_async_copy(
                    rows_v.at[b], accum.at[dst_v.at[0]], ssems[b]).wait()

            _gather(0, 0).start()

            def _step(i, carry):
                k = i * 2
                for b in range(2):
                    cur = k + b

                    _gather(cur, b).wait()
                    pltpu.async_copy(rows_v.at[b],
                                     accum.at[dst_v.at[cur]], ssems[b],
                                     add=True)

                    @pl.when(cur + 1 < NCH)
                    def _():
                        @pl.when(cur >= 1)
                        def _():
                            _scatter_drain(1 - b)
                        _gather(cur + 1, 1 - b).start()
                return carry

            lax.fori_loop(0, NCH // 2, _step, 0)
            _scatter_drain(0)
            _scatter_drain(1)
            plsc.subcore_barrier()
            pltpu.sync_copy(
                accum.at[pl.ds(s * FPT, FPT)],
                out_hbm.at[g, t, pl.ds(c * NHALF + s * FPT, FPT)])


def _sc_spmm(table, src_idx, dst_idx, zeros):
    mesh = plsc.VectorSubcoreMesh(core_axis_name="c", subcore_axis_name="s")
    return pl.kernel(
        _spmm_body,
        out_type=jax.ShapeDtypeStruct((NBG, 2, NP, HID), jnp.float32),
        mesh=mesh,
        scratch_types=[
            pltpu.VMEM((NCH, CH), jnp.int32),
            pltpu.VMEM((NCH, CH), jnp.int32),
            pltpu.VMEM((2, CH, HID), jnp.float32),
            pltpu.VMEM_SHARED((ACC_R, HID), jnp.float32),
            pltpu.SemaphoreType.DMA,
            pltpu.SemaphoreType.DMA,
            pltpu.SemaphoreType.DMA,
            pltpu.SemaphoreType.DMA,
        ],
    )(table, src_idx, dst_idx, zeros)


def _post_body(agg_ref, b_ref, w_ref, bd_ref, sb1_ref, sb2_ref, out_ref):
    bd = bd_ref[0, 0]
    sb1 = sb1_ref[0, 0]
    sb2 = sb2_ref[0, 0]
    for g in range(NBG):
        h1 = jnp.maximum(agg_ref[g, 0] + b_ref[g][None, :], 0.0)
        h2 = jnp.maximum(agg_ref[g, 1] + b_ref[g][None, :], 0.0)
        m = jnp.mean(h1, axis=0)
        c = 1.0 / (1.0 + jnp.exp(-m))
        u = jnp.dot(w_ref[...], c[:, None],
                    preferred_element_type=jnp.float32)
        sc1 = jnp.dot(h1, u, preferred_element_type=jnp.float32)[:, 0]
        sc2 = jnp.dot(h2, u, preferred_element_type=jnp.float32)[:, 0]
        out_ref[g, 0, :] = sc1 + bd + sb1
        out_ref[g, 1, :] = sc2 + bd + sb2


def _post(agg, b_gcn, w_disc, bd, sb1, sb2):
    return pl.pallas_call(
        _post_body,
        out_shape=jax.ShapeDtypeStruct((NBG, 2, N), jnp.float32),
    )(agg, b_gcn, w_disc, bd, sb1, sb2)


def kernel(feature, adj, shuf, sparse, msk, samp_bias1, samp_bias2,
           W_gcn, b_gcn, W_disc, b_disc):
    tables = _gcn_matmul(feature, shuf, W_gcn).reshape(4 * N, HID)

    offs = (2 * jnp.arange(NBG, dtype=jnp.int32) * N)[:, None, None] \
        + (jnp.arange(2, dtype=jnp.int32) * N)[None, :, None]
    src_idx = (adj[:, 0, None, :] + offs).reshape(NBG, 2, NTILES, NCH, CH)
    dst = adj[:, 1, :]
    trash = NHALF + (dst & 127)
    dst_c = jnp.stack([
        jnp.where(dst < NHALF, dst, trash),
        jnp.where(dst >= NHALF, dst - NHALF, trash),
    ]).reshape(2, NBG, NTILES, NCH, CH)
    zeros = jnp.zeros((ZPT, HID), dtype=jnp.float32)

    agg = _sc_spmm(tables, src_idx, dst_c, zeros)[:, :, :N, :]

    bd = jnp.reshape(b_disc, (1, 1)).astype(jnp.float32)
    sb1 = jnp.reshape(samp_bias1, (1, 1)).astype(jnp.float32)
    sb2 = jnp.reshape(samp_bias2, (1, 1)).astype(jnp.float32)
    sc = _post(agg, b_gcn, W_disc, bd, sb1, sb2).reshape(NBG, 2 * N)

    out = jnp.stack([sc[0], sc[1], sc[1], sc[0]])[:, None, :]
    return out


# speedup vs baseline: 1.3109x; 1.3109x over previous
import jax
import jax.numpy as jnp
from jax import lax
from jax.experimental import pallas as pl
from jax.experimental.pallas import tpu as pltpu
from jax.experimental.pallas import tpu_sc as plsc

NBG = 2
N = 10000
FT = 128
HID = 128
E = 320000

NTILES = 16
EPT = E // NTILES
CH = 80
NCH = EPT // CH
NP = 10240
NHALF = NP // 2
ACC_R = NHALF + 128
ZPT = ACC_R // NTILES
FPT = NHALF // NTILES


def _mm_body(f_ref, s_ref, w_ref, o_ref):
    w = w_ref[0]
    o_ref[0] = jnp.dot(f_ref[0, 0], w, preferred_element_type=jnp.float32)
    o_ref[1] = jnp.dot(s_ref[0, 0], w, preferred_element_type=jnp.float32)


def _gcn_matmul(feature, shuf, w):
    return pl.pallas_call(
        _mm_body,
        grid=(NBG,),
        in_specs=[
            pl.BlockSpec((1, 1, N, FT), lambda g: (g, 0, 0, 0)),
            pl.BlockSpec((1, 1, N, FT), lambda g: (g, 0, 0, 0)),
            pl.BlockSpec((1, FT, HID), lambda g: (g, 0, 0)),
        ],
        out_specs=pl.BlockSpec((2, N, HID), lambda g: (g, 0, 0)),
        out_shape=jax.ShapeDtypeStruct((4, N, HID), jnp.float32),
    )(feature, shuf, w)


def _spmm_body(table_hbm, src_hbm, dst_hbm, zeros_hbm, out_hbm,
               src_v, dst_v, rows_v, accum, sem0, sem1):
    c = lax.axis_index("c")
    s = lax.axis_index("s")
    sems = (sem0, sem1)

    for g in range(NBG):
        for t in range(2):
            pltpu.sync_copy(zeros_hbm, accum.at[pl.ds(s * ZPT, ZPT)])
            pltpu.sync_copy(src_hbm.at[g, t, s], src_v)
            pltpu.sync_copy(dst_hbm.at[c, g, s], dst_v)
            plsc.subcore_barrier()

            def _gather(k, b):
                return pltpu.make_async_copy(
                    table_hbm.at[src_v.at[k]], rows_v.at[b], sems[b])

            _gather(0, 0).start()

            def _step(i, carry):
                k = i * 2
                for b in range(2):
                    cur = k + b

                    @pl.when(cur + 1 < NCH)
                    def _():
                        _gather(cur + 1, 1 - b).start()

                    _gather(cur, b).wait()
                    pltpu.sync_copy(rows_v.at[b],
                                    accum.at[dst_v.at[cur]], add=True)
                return carry

            lax.fori_loop(0, NCH // 2, _step, 0)
            plsc.subcore_barrier()
            pltpu.sync_copy(
                accum.at[pl.ds(s * FPT, FPT)],
                out_hbm.at[g, t, pl.ds(c * NHALF + s * FPT, FPT)])


def _sc_spmm(table, src_idx, dst_idx, zeros):
    mesh = plsc.VectorSubcoreMesh(core_axis_name="c", subcore_axis_name="s")
    return pl.kernel(
        _spmm_body,
        out_type=jax.ShapeDtypeStruct((NBG, 2, NP, HID), jnp.float32),
        mesh=mesh,
        scratch_types=[
            pltpu.VMEM((NCH, CH), jnp.int32),
            pltpu.VMEM((NCH, CH), jnp.int32),
            pltpu.VMEM((2, CH, HID), jnp.float32),
            pltpu.VMEM_SHARED((ACC_R, HID), jnp.float32),
            pltpu.SemaphoreType.DMA,
            pltpu.SemaphoreType.DMA,
        ],
    )(table, src_idx, dst_idx, zeros)


def _post_body(agg_ref, b_ref, w_ref, bd_ref, sb1_ref, sb2_ref, out_ref):
    bd = bd_ref[0, 0]
    sb1 = sb1_ref[0, 0]
    sb2 = sb2_ref[0, 0]
    for g in range(NBG):
        h1 = jnp.maximum(agg_ref[g, 0] + b_ref[g][None, :], 0.0)
        h2 = jnp.maximum(agg_ref[g, 1] + b_ref[g][None, :], 0.0)
        m = jnp.mean(h1, axis=0)
        c = 1.0 / (1.0 + jnp.exp(-m))
        u = jnp.dot(w_ref[...], c[:, None],
                    preferred_element_type=jnp.float32)
        sc1 = jnp.dot(h1, u, preferred_element_type=jnp.float32)[:, 0]
        sc2 = jnp.dot(h2, u, preferred_element_type=jnp.float32)[:, 0]
        out_ref[g, 0, :] = sc1 + bd + sb1
        out_ref[g, 1, :] = sc2 + bd + sb2


def _post(agg, b_gcn, w_disc, bd, sb1, sb2):
    return pl.pallas_call(
        _post_body,
        out_shape=jax.ShapeDtypeStruct((NBG, 2, N), jnp.float32),
    )(agg, b_gcn, w_disc, bd, sb1, sb2)


def kernel(feature, adj, shuf, sparse, msk, samp_bias1, samp_bias2,
           W_gcn, b_gcn, W_disc, b_disc):
    tables = _gcn_matmul(feature, shuf, W_gcn).reshape(4 * N, HID)

    offs = (2 * jnp.arange(NBG, dtype=jnp.int32) * N)[:, None, None] \
        + (jnp.arange(2, dtype=jnp.int32) * N)[None, :, None]
    src_idx = (adj[:, 0, None, :] + offs).reshape(NBG, 2, NTILES, NCH, CH)
    dst = adj[:, 1, :]
    trash = NHALF + (dst & 127)
    dst_c = jnp.stack([
        jnp.where(dst < NHALF, dst, trash),
        jnp.where(dst >= NHALF, dst - NHALF, trash),
    ]).reshape(2, NBG, NTILES, NCH, CH)
    zeros = jnp.zeros((ZPT, HID), dtype=jnp.float32)

    agg = _sc_spmm(tables, src_idx, dst_c, zeros)[:, :, :N, :]

    bd = jnp.reshape(b_disc, (1, 1)).astype(jnp.float32)
    sb1 = jnp.reshape(samp_bias1, (1, 1)).astype(jnp.float32)
    sb2 = jnp.reshape(samp_bias2, (1, 1)).astype(jnp.float32)
    sc = _post(agg, b_gcn, W_disc, bd, sb1, sb2).reshape(NBG, 2 * N)

    out = jnp.stack([sc[0], sc[1], sc[1], sc[0]])[:, None, :]
    return out


# speedup vs baseline: 1.3253x; 1.0109x over previous
import jax
import jax.numpy as jnp
from jax import lax
from jax.experimental import pallas as pl
from jax.experimental.pallas import tpu as pltpu
from jax.experimental.pallas import tpu_sc as plsc

NBG = 2
N = 10000
FT = 128
HID = 128
E = 320000

NTILES = 16
EPT = E // NTILES
CH = 80
NCH = EPT // CH
NP = 10240
NHALF = NP // 2
ACC_R = NHALF + 128
ZPT = ACC_R // NTILES
FPT = NHALF // NTILES


def _mm_body(f_ref, s_ref, w_ref, o_ref):
    w = w_ref[0]
    o_ref[0] = jnp.dot(f_ref[0, 0], w, preferred_element_type=jnp.float32)
    o_ref[1] = jnp.dot(s_ref[0, 0], w, preferred_element_type=jnp.float32)


def _gcn_matmul(feature, shuf, w):
    return pl.pallas_call(
        _mm_body,
        grid=(NBG,),
        in_specs=[
            pl.BlockSpec((1, 1, N, FT), lambda g: (g, 0, 0, 0)),
            pl.BlockSpec((1, 1, N, FT), lambda g: (g, 0, 0, 0)),
            pl.BlockSpec((1, FT, HID), lambda g: (g, 0, 0)),
        ],
        out_specs=pl.BlockSpec((2, N, HID), lambda g: (g, 0, 0)),
        out_shape=jax.ShapeDtypeStruct((4, N, HID), jnp.float32),
    )(feature, shuf, w)


def _spmm_body(table_hbm, src_hbm, dst_hbm, zeros_hbm, out_hbm,
               src_v, dst_v, rows_v, accum, sem0, sem1):
    c = lax.axis_index("c")
    s = lax.axis_index("s")
    sems = (sem0, sem1)

    for g in range(NBG):
        for t in range(2):
            def _zero():
                return pltpu.make_async_copy(
                    zeros_hbm.at[pl.ds(0, FPT)],
                    accum.at[pl.ds(s * FPT, FPT)], sems[1])

            def _zero_trash():
                return pltpu.make_async_copy(
                    zeros_hbm.at[pl.ds(0, ACC_R - NHALF)],
                    accum.at[pl.ds(NHALF, ACC_R - NHALF)], sems[1])

            def _src_load():
                return pltpu.make_async_copy(src_hbm.at[g, t, s], src_v,
                                             sems[0])

            def _gather(k, b):
                return pltpu.make_async_copy(
                    table_hbm.at[src_v.at[k]], rows_v.at[b], sems[b])

            _zero().start()

            @pl.when(s == 0)
            def _():
                _zero_trash().start()

            _src_load().start()
            pltpu.sync_copy(dst_hbm.at[c, g, s], dst_v)
            _src_load().wait()
            _gather(0, 0).start()
            _zero().wait()

            @pl.when(s == 0)
            def _():
                _zero_trash().wait()

            plsc.subcore_barrier()

            def _step(i, carry):
                k = i * 2
                for b in range(2):
                    cur = k + b

                    @pl.when(cur + 1 < NCH)
                    def _():
                        _gather(cur + 1, 1 - b).start()

                    _gather(cur, b).wait()
                    pltpu.sync_copy(rows_v.at[b],
                                    accum.at[dst_v.at[cur]], add=True)
                return carry

            lax.fori_loop(0, NCH // 2, _step, 0)
            plsc.subcore_barrier()
            pltpu.sync_copy(
                accum.at[pl.ds(s * FPT, FPT)],
                out_hbm.at[g, t, pl.ds(c * NHALF + s * FPT, FPT)])


def _sc_spmm(table, src_idx, dst_idx, zeros):
    mesh = plsc.VectorSubcoreMesh(core_axis_name="c", subcore_axis_name="s")
    return pl.kernel(
        _spmm_body,
        out_type=jax.ShapeDtypeStruct((NBG, 2, NP, HID), jnp.float32),
        mesh=mesh,
        scratch_types=[
            pltpu.VMEM((NCH, CH), jnp.int32),
            pltpu.VMEM((NCH, CH), jnp.int32),
            pltpu.VMEM((2, CH, HID), jnp.float32),
            pltpu.VMEM_SHARED((ACC_R, HID), jnp.float32),
            pltpu.SemaphoreType.DMA,
            pltpu.SemaphoreType.DMA,
        ],
    )(table, src_idx, dst_idx, zeros)


def _post_body(agg_ref, b_ref, w_ref, bd_ref, sb1_ref, sb2_ref, out_ref):
    bd = bd_ref[0, 0]
    sb1 = sb1_ref[0, 0]
    sb2 = sb2_ref[0, 0]
    for g in range(NBG):
        h1 = jnp.maximum(agg_ref[g, 0] + b_ref[g][None, :], 0.0)
        h2 = jnp.maximum(agg_ref[g, 1] + b_ref[g][None, :], 0.0)
        m = jnp.mean(h1, axis=0)
        c = 1.0 / (1.0 + jnp.exp(-m))
        u = jnp.dot(w_ref[...], c[:, None],
                    preferred_element_type=jnp.float32)
        sc1 = jnp.dot(h1, u, preferred_element_type=jnp.float32)[:, 0]
        sc2 = jnp.dot(h2, u, preferred_element_type=jnp.float32)[:, 0]
        out_ref[g, 0, :] = sc1 + bd + sb1
        out_ref[g, 1, :] = sc2 + bd + sb2


def _post(agg, b_gcn, w_disc, bd, sb1, sb2):
    return pl.pallas_call(
        _post_body,
        out_shape=jax.ShapeDtypeStruct((NBG, 2, N), jnp.float32),
    )(agg, b_gcn, w_disc, bd, sb1, sb2)


def kernel(feature, adj, shuf, sparse, msk, samp_bias1, samp_bias2,
           W_gcn, b_gcn, W_disc, b_disc):
    tables = _gcn_matmul(feature, shuf, W_gcn).reshape(4 * N, HID)

    offs = (2 * jnp.arange(NBG, dtype=jnp.int32) * N)[:, None, None] \
        + (jnp.arange(2, dtype=jnp.int32) * N)[None, :, None]
    src_idx = (adj[:, 0, None, :] + offs).reshape(NBG, 2, NTILES, NCH, CH)
    dst = adj[:, 1, :]
    trash = NHALF + (dst & 127)
    dst_c = jnp.stack([
        jnp.where(dst < NHALF, dst, trash),
        jnp.where(dst >= NHALF, dst - NHALF, trash),
    ]).reshape(2, NBG, NTILES, NCH, CH)
    zeros = jnp.zeros((ZPT, HID), dtype=jnp.float32)

    agg = _sc_spmm(tables, src_idx, dst_c, zeros)[:, :, :N, :]

    bd = jnp.reshape(b_disc, (1, 1)).astype(jnp.float32)
    sb1 = jnp.reshape(samp_bias1, (1, 1)).astype(jnp.float32)
    sb2 = jnp.reshape(samp_bias2, (1, 1)).astype(jnp.float32)
    sc = _post(agg, b_gcn, W_disc, bd, sb1, sb2).reshape(NBG, 2 * N)

    out = jnp.stack([sc[0], sc[1], sc[1], sc[0]])[:, None, :]
    return out
